# Initial kernel scaffold; baseline (speedup 1.0000x reference)
import jax
import jax.numpy as jnp
from jax.experimental import pallas as pl


def kernel(hidden_states, W, b):
    raise NotImplementedError("write your pallas kernel here")


# speedup vs baseline: 1.0777x; 1.0777x over previous
import functools

import jax
import jax.numpy as jnp
from jax.experimental import pallas as pl
from jax.experimental.pallas import tpu as pltpu

B, S, D, E, K = 2, 4096, 4096, 64, 8
S_TILE = 128
N_TILES = S // S_TILE


def _router_body(h_ref, w_ref, b_ref, ow_ref, oi_ref, os_ref, acc_ref):
    i = pl.program_id(0)

    @pl.when(i == 0)
    def _init():
        acc_ref[...] = jnp.zeros_like(acc_ref)

    acc_ref[...] += jnp.sum(h_ref[...], axis=1)

    @pl.when(i == N_TILES - 1)
    def _epilogue():
        pooled = acc_ref[...] * (1.0 / S)
        logits = jax.lax.dot_general(
            pooled, w_ref[...], (((1,), (1,)), ((), ())),
            preferred_element_type=jnp.float32) + b_ref[...]
        m = jnp.max(logits, axis=-1, keepdims=True)
        ex = jnp.exp(logits - m)
        sumex = jnp.sum(ex, axis=-1, keepdims=True)
        probs = ex / sumex

        iota = jax.lax.broadcasted_iota(jnp.int32, (B, E), 1)
        masked = probs
        sel_mask = jnp.zeros((B, E), jnp.float32)
        vals, idxs = [], []
        for _ in range(K):
            v = jnp.max(masked, axis=-1, keepdims=True)
            is_max = masked == v
            idx = jnp.min(jnp.where(is_max, iota, E), axis=-1,
                          keepdims=True)
            chosen = iota == idx
            sel_mask = sel_mask + chosen.astype(jnp.float32)
            vals.append(v)
            idxs.append(idx)
            masked = jnp.where(chosen, -1.0, masked)
        top_vals = jnp.concatenate(vals, axis=1)
        top_idx = jnp.concatenate(idxs, axis=1)
        top_w = top_vals / jnp.sum(top_vals, axis=-1, keepdims=True)

        f = jnp.mean(sel_mask, axis=0)
        p_mean = jnp.mean(probs, axis=0)
        lb = E * jnp.sum(f * p_mean)
        p_sq = jnp.mean(probs * probs, axis=0)
        aux = jnp.mean((p_sq - 1.0 / E) ** 2)
        lse = m[:, 0] + jnp.log(sumex[:, 0])
        z = jnp.mean(lse * lse)

        ow_ref[...] = jnp.pad(top_w, ((0, 8 - B), (0, 128 - K)))
        oi_ref[...] = jnp.pad(top_idx, ((0, 8 - B), (0, 128 - K)))
        scal = jnp.concatenate(
            [lb.reshape(1, 1), aux.reshape(1, 1), z.reshape(1, 1)], axis=1)
        os_ref[...] = jnp.pad(scal, ((0, 7), (0, 125)))


@jax.jit
def kernel(hidden_states, W, b):
    ow, oi, osc = pl.pallas_call(
        _router_body,
        grid=(N_TILES,),
        in_specs=[
            pl.BlockSpec((B, S_TILE, D), lambda i: (0, i, 0)),
            pl.BlockSpec((E, D), lambda i: (0, 0)),
            pl.BlockSpec((1, E), lambda i: (0, 0)),
        ],
        out_specs=[
            pl.BlockSpec((8, 128), lambda i: (0, 0)),
            pl.BlockSpec((8, 128), lambda i: (0, 0)),
            pl.BlockSpec((8, 128), lambda i: (0, 0)),
        ],
        out_shape=[
            jax.ShapeDtypeStruct((8, 128), jnp.float32),
            jax.ShapeDtypeStruct((8, 128), jnp.int32),
            jax.ShapeDtypeStruct((8, 128), jnp.float32),
        ],
        scratch_shapes=[pltpu.VMEM((B, D), jnp.float32)],
    )(hidden_states, W, b.reshape(1, E))
    top_k_indices = oi[:B, :K]
    top_k_weights = ow[:B, :K]
    return (top_k_indices, top_k_weights, osc[0, 0], osc[0, 1], osc[0, 2])


# speedup vs baseline: 1.1001x; 1.0207x over previous
import functools

import jax
import jax.numpy as jnp
from jax.experimental import pallas as pl
from jax.experimental.pallas import tpu as pltpu

B, S, D, E, K = 2, 4096, 4096, 64, 8
S_TILE = 512
N_TILES = S // S_TILE


def _router_body(h_ref, w_ref, b_ref, ow_ref, oi_ref, os_ref, acc_ref):
    i = pl.program_id(0)

    @pl.when(i == 0)
    def _init():
        acc_ref[...] = jnp.zeros_like(acc_ref)

    acc_ref[...] += jnp.sum(h_ref[...], axis=1)

    @pl.when(i == N_TILES - 1)
    def _epilogue():
        pooled = acc_ref[...] * (1.0 / S)
        logits = jax.lax.dot_general(
            pooled, w_ref[...], (((1,), (1,)), ((), ())),
            preferred_element_type=jnp.float32) + b_ref[...]
        m = jnp.max(logits, axis=-1, keepdims=True)
        ex = jnp.exp(logits - m)
        sumex = jnp.sum(ex, axis=-1, keepdims=True)
        probs = ex / sumex

        iota = jax.lax.broadcasted_iota(jnp.int32, (B, E), 1)
        masked = probs
        sel_mask = jnp.zeros((B, E), jnp.float32)
        vals, idxs = [], []
        for _ in range(K):
            v = jnp.max(masked, axis=-1, keepdims=True)
            is_max = masked == v
            idx = jnp.min(jnp.where(is_max, iota, E), axis=-1,
                          keepdims=True)
            chosen = iota == idx
            sel_mask = sel_mask + chosen.astype(jnp.float32)
            vals.append(v)
            idxs.append(idx)
            masked = jnp.where(chosen, -1.0, masked)
        top_vals = jnp.concatenate(vals, axis=1)
        top_idx = jnp.concatenate(idxs, axis=1)
        top_w = top_vals / jnp.sum(top_vals, axis=-1, keepdims=True)

        f = jnp.mean(sel_mask, axis=0)
        p_mean = jnp.mean(probs, axis=0)
        lb = E * jnp.sum(f * p_mean)
        p_sq = jnp.mean(probs * probs, axis=0)
        aux = jnp.mean((p_sq - 1.0 / E) ** 2)
        lse = m[:, 0] + jnp.log(sumex[:, 0])
        z = jnp.mean(lse * lse)

        ow_ref[...] = jnp.pad(top_w, ((0, 8 - B), (0, 128 - K)))
        oi_ref[...] = jnp.pad(top_idx, ((0, 8 - B), (0, 128 - K)))
        scal = jnp.concatenate(
            [lb.reshape(1, 1), aux.reshape(1, 1), z.reshape(1, 1)], axis=1)
        os_ref[...] = jnp.pad(scal, ((0, 7), (0, 125)))


@jax.jit
def kernel(hidden_states, W, b):
    ow, oi, osc = pl.pallas_call(
        _router_body,
        grid=(N_TILES,),
        in_specs=[
            pl.BlockSpec((B, S_TILE, D), lambda i: (0, i, 0)),
            pl.BlockSpec((E, D), lambda i: (0, 0)),
            pl.BlockSpec((1, E), lambda i: (0, 0)),
        ],
        out_specs=[
            pl.BlockSpec((8, 128), lambda i: (0, 0)),
            pl.BlockSpec((8, 128), lambda i: (0, 0)),
            pl.BlockSpec((8, 128), lambda i: (0, 0)),
        ],
        out_shape=[
            jax.ShapeDtypeStruct((8, 128), jnp.float32),
            jax.ShapeDtypeStruct((8, 128), jnp.int32),
            jax.ShapeDtypeStruct((8, 128), jnp.float32),
        ],
        scratch_shapes=[pltpu.VMEM((B, D), jnp.float32)],
    )(hidden_states, W, b.reshape(1, E))
    top_k_indices = oi[:B, :K]
    top_k_weights = ow[:B, :K]
    return (top_k_indices, top_k_weights, osc[0, 0], osc[0, 1], osc[0, 2])
